# Initial kernel scaffold; baseline (speedup 1.0000x reference)
import jax
import jax.numpy as jnp
from jax.experimental import pallas as pl


def kernel(features, edge_index, W1, b1, W2, b2):
    raise NotImplementedError("write your pallas kernel here")


# speedup vs baseline: 17.8855x; 17.8855x over previous
import functools

import jax
import jax.numpy as jnp
from jax import lax
from jax.experimental import pallas as pl
from jax.experimental.pallas import tpu as pltpu
from jax.experimental.pallas import tpu_sc as plsc

N = 10000
E = 320000
D = 128
H = 128
C = 16

NC = 2
NS = 16
NW = NC * NS
EPT = E // NW
RPT = 640

DW, DB = 5, 2000
NPAD = NS * 640
AW, AB = 10, 1000

_mesh = plsc.VectorSubcoreMesh(core_axis_name="c", subcore_axis_name="s")
_f32 = jnp.float32


@functools.partial(
    pl.kernel,
    out_type=(
        jax.ShapeDtypeStruct((NW * 640,), _f32),
        jax.ShapeDtypeStruct((NW * 640,), _f32),
    ),
    mesh=_mesh,
    scratch_types=[
        [pltpu.VMEM((DB,), jnp.int32) for _ in range(DW)],
        [pltpu.VMEM((DB,), jnp.int32) for _ in range(DW)],
        pltpu.VMEM((DB,), _f32),
        pltpu.VMEM((640,), _f32),
        pltpu.VMEM_SHARED((NPAD,), _f32),
        pltpu.VMEM_SHARED((NPAD,), _f32),
    ],
)
def _deg(src_hbm, dst_hbm, outs_hbm, outd_hbm,
         sidx_v, didx_v, ones_v, tmp_v, degs_sh, degd_sh):
    c = lax.axis_index("c")
    s = lax.axis_index("s")
    wid = c * NS + s
    ebase = wid * EPT

    def frow(i, carry):
        ones_v[pl.ds(pl.multiple_of(i * 16, 16), 16)] = jnp.ones((16,), _f32)
        return carry

    lax.fori_loop(0, DB // 16, frow, 0)

    def zrow(i, carry):
        tmp_v[pl.ds(pl.multiple_of(i * 16, 16), 16)] = jnp.zeros((16,), _f32)
        return carry

    lax.fori_loop(0, 640 // 16, zrow, 0)
    pltpu.sync_copy(tmp_v, degs_sh.at[pl.ds(s * 640, 640)])
    pltpu.sync_copy(tmp_v, degd_sh.at[pl.ds(s * 640, 640)])
    for w in range(DW):
        pltpu.sync_copy(src_hbm.at[pl.ds(ebase + w * DB, DB)], sidx_v[w])
        pltpu.sync_copy(dst_hbm.at[pl.ds(ebase + w * DB, DB)], didx_v[w])
    plsc.subcore_barrier()
    for w in range(DW):
        pltpu.sync_copy(ones_v, degs_sh.at[sidx_v[w]], add=True)
        pltpu.sync_copy(ones_v, degd_sh.at[didx_v[w]], add=True)
    plsc.subcore_barrier()
    pltpu.sync_copy(degs_sh.at[pl.ds(s * 640, 640)], tmp_v)
    pltpu.sync_copy(tmp_v, outs_hbm.at[pl.ds(wid * 640, 640)])
    pltpu.sync_copy(degd_sh.at[pl.ds(s * 640, 640)], tmp_v)
    pltpu.sync_copy(tmp_v, outd_hbm.at[pl.ds(wid * 640, 640)])


@functools.partial(
    pl.kernel,
    out_type=jax.ShapeDtypeStruct((NC, NS * RPT, C), _f32),
    mesh=_mesh,
    compiler_params=pltpu.CompilerParams(use_tc_tiling_on_sc=False),
    scratch_types=[
        [pltpu.VMEM((AB,), jnp.int32) for _ in range(AW)],
        [pltpu.VMEM((AB,), jnp.int32) for _ in range(AW)],
        pltpu.VMEM((AB, C), _f32),
        pltpu.VMEM((RPT, C), _f32),
        pltpu.VMEM_SHARED((NS * RPT, C), _f32),
        pltpu.SemaphoreType.DMA,
    ],
)
def _agg(src_hbm, dst_hbm, p_hbm, out_hbm,
         sidx_v, didx_v, rows_v, tmp_v, acc_sh, sem):
    c = lax.axis_index("c")
    s = lax.axis_index("s")
    wid = c * NS + s
    ebase = wid * EPT

    def zrow(i, carry):
        tmp_v[i, :] = jnp.zeros((C,), _f32)
        return carry

    lax.fori_loop(0, RPT, zrow, 0)
    pltpu.sync_copy(tmp_v, acc_sh.at[pl.ds(s * RPT, RPT)])
    for w in range(AW):
        pltpu.sync_copy(src_hbm.at[pl.ds(ebase + w * AB, AB)], sidx_v[w])
        pltpu.sync_copy(dst_hbm.at[pl.ds(ebase + w * AB, AB)], didx_v[w])
    plsc.subcore_barrier()
    for w in range(AW):
        pltpu.async_copy(p_hbm.at[sidx_v[w]], rows_v, sem).wait()
        pltpu.sync_copy(rows_v, acc_sh.at[didx_v[w]], add=True)
    plsc.subcore_barrier()
    pltpu.sync_copy(acc_sh.at[pl.ds(s * RPT, RPT)], tmp_v)
    pltpu.sync_copy(tmp_v, out_hbm.at[c, pl.ds(s * RPT, RPT)])


def _tc1_body(x_ref, w1_ref, w2_ref, b1_ref, ds_ref, dd_ref,
              p_ref, dsrc_ref, ddst_ref, bc_ref):
    wc = jnp.dot(w1_ref[...], w2_ref[...], preferred_element_type=_f32)
    p = jnp.dot(x_ref[...], wc, preferred_element_type=_f32)
    dsum_s = ds_ref[0] + ds_ref[1]
    dsum_d = dd_ref[0] + dd_ref[1]
    dsrc = lax.rsqrt(jnp.maximum(dsum_s, 1.0))
    ddst = lax.rsqrt(jnp.maximum(dsum_d, 1.0))
    p_ref[...] = p * dsrc
    dsrc_ref[...] = dsrc
    ddst_ref[...] = ddst
    bc_ref[...] = jnp.dot(b1_ref[...], w2_ref[...], preferred_element_type=_f32)


_tc1 = pl.pallas_call(
    _tc1_body,
    out_shape=(
        jax.ShapeDtypeStruct((N, C), _f32),
        jax.ShapeDtypeStruct((N, 1), _f32),
        jax.ShapeDtypeStruct((N, 1), _f32),
        jax.ShapeDtypeStruct((1, C), _f32),
    ),
)


def _tc2_body(acc_ref, ddst_ref, dsrc_ref, bc_ref, out_ref):
    q = (acc_ref[0] + acc_ref[1]) * ddst_ref[...]
    out_ref[...] = (q + bc_ref[...]) * dsrc_ref[...]


_tc2 = pl.pallas_call(
    _tc2_body,
    out_shape=jax.ShapeDtypeStruct((N, C), _f32),
)


def _tc3_body(acc_ref, ddst_ref, b2_ref, out_ref):
    out_ref[...] = (acc_ref[0] + acc_ref[1]) * ddst_ref[...] + b2_ref[...]


_tc3 = pl.pallas_call(
    _tc3_body,
    out_shape=jax.ShapeDtypeStruct((N, C), _f32),
)


def kernel(features, edge_index, W1, b1, W2, b2):
    src = edge_index[0]
    dst = edge_index[1]

    degs_p, degd_p = _deg(src, dst)
    degs = degs_p.reshape(NC, NS * 640)[:, :N].reshape(NC, N, 1)
    degd = degd_p.reshape(NC, NS * 640)[:, :N].reshape(NC, N, 1)

    pscaled, dsrc, ddst, bc = _tc1(
        features, W1, W2, b1.reshape(1, H), degs, degd)

    acc1 = _agg(src, dst, pscaled)[:, :N, :]
    rs = _tc2(acc1, ddst, dsrc, bc)
    acc2 = _agg(src, dst, rs)[:, :N, :]
    return _tc3(acc2, ddst, b2.reshape(1, C))


# speedup vs baseline: 24.0780x; 1.3462x over previous
import functools

import jax
import jax.numpy as jnp
from jax import lax
from jax.experimental import pallas as pl
from jax.experimental.pallas import tpu as pltpu
from jax.experimental.pallas import tpu_sc as plsc

N = 10000
E = 320000
D = 128
H = 128
C = 16

NC = 2
NS = 16
NW = NC * NS
EPT = E // NW
RPT = 640
NPAD = NS * RPT

DW, DB = 5, 2000
AW, AB = 10, 1000

_mesh = plsc.VectorSubcoreMesh(core_axis_name="c", subcore_axis_name="s")
_f32 = jnp.float32
_sc_params = pltpu.CompilerParams(use_tc_tiling_on_sc=False, needs_layout_passes=False)


def _rsqrt16(x):
    bits = plsc.bitcast(x, jnp.int32)
    y = plsc.bitcast(jnp.int32(0x5F3759DF) - (bits >> 1), _f32)
    for _ in range(3):
        y = y * (1.5 - 0.5 * x * y * y)
    return y


@functools.partial(
    pl.kernel,
    out_type=(
        jax.ShapeDtypeStruct((NW * RPT,), _f32),
        jax.ShapeDtypeStruct((NW * RPT,), _f32),
    ),
    mesh=_mesh,
    scratch_types=[
        [pltpu.VMEM((DB,), jnp.int32) for _ in range(DW)],
        [pltpu.VMEM((DB,), jnp.int32) for _ in range(DW)],
        pltpu.VMEM((DB,), _f32),
        pltpu.VMEM((RPT,), _f32),
        pltpu.VMEM_SHARED((NPAD,), _f32),
        pltpu.VMEM_SHARED((NPAD,), _f32),
    ],
)
def _deg(src_hbm, dst_hbm, outs_hbm, outd_hbm,
         sidx_v, didx_v, ones_v, tmp_v, degs_sh, degd_sh):
    c = lax.axis_index("c")
    s = lax.axis_index("s")
    wid = c * NS + s
    ebase = wid * EPT

    def frow(i, carry):
        ones_v[pl.ds(pl.multiple_of(i * 16, 16), 16)] = jnp.ones((16,), _f32)
        return carry

    lax.fori_loop(0, DB // 16, frow, 0)

    def zrow(i, carry):
        tmp_v[pl.ds(pl.multiple_of(i * 16, 16), 16)] = jnp.zeros((16,), _f32)
        return carry

    lax.fori_loop(0, RPT // 16, zrow, 0)
    pltpu.sync_copy(tmp_v, degs_sh.at[pl.ds(s * RPT, RPT)])
    pltpu.sync_copy(tmp_v, degd_sh.at[pl.ds(s * RPT, RPT)])
    for w in range(DW):
        pltpu.sync_copy(src_hbm.at[pl.ds(ebase + w * DB, DB)], sidx_v[w])
        pltpu.sync_copy(dst_hbm.at[pl.ds(ebase + w * DB, DB)], didx_v[w])
    plsc.subcore_barrier()
    for w in range(DW):
        pltpu.sync_copy(ones_v, degs_sh.at[sidx_v[w]], add=True)
        pltpu.sync_copy(ones_v, degd_sh.at[didx_v[w]], add=True)
    plsc.subcore_barrier()
    pltpu.sync_copy(degs_sh.at[pl.ds(s * RPT, RPT)], tmp_v)
    pltpu.sync_copy(tmp_v, outs_hbm.at[pl.ds(wid * RPT, RPT)])
    pltpu.sync_copy(degd_sh.at[pl.ds(s * RPT, RPT)], tmp_v)
    pltpu.sync_copy(tmp_v, outd_hbm.at[pl.ds(wid * RPT, RPT)])


def _agg_main(src_hbm, dst_hbm, out_hbm, sidx_v, didx_v, rows_v, tmp_v,
              ptab_sh, acc_sh, gsem, ssem, c, s):
    wid = c * NS + s
    ebase = wid * EPT

    def zrow(i, carry):
        tmp_v[i, :] = jnp.zeros((C,), _f32)
        return carry

    lax.fori_loop(0, RPT, zrow, 0)
    pltpu.sync_copy(tmp_v, acc_sh.at[pl.ds(s * RPT, RPT)])
    for w in range(AW):
        pltpu.sync_copy(src_hbm.at[pl.ds(ebase + w * AB, AB)], sidx_v[w])
        pltpu.sync_copy(dst_hbm.at[pl.ds(ebase + w * AB, AB)], didx_v[w])
    plsc.subcore_barrier()

    g_desc = [None] * AW
    s_desc = [None] * AW
    g_desc[0] = pltpu.async_copy(ptab_sh.at[sidx_v[0]], rows_v[0], gsem[0])
    for w in range(AW):
        g_desc[w].wait()
        s_desc[w] = pltpu.async_copy(
            rows_v[w % 2], acc_sh.at[didx_v[w]], ssem[w % 2], add=True)
        if w + 1 < AW:
            if w >= 1:
                s_desc[w - 1].wait()
            g_desc[w + 1] = pltpu.async_copy(
                ptab_sh.at[sidx_v[w + 1]], rows_v[(w + 1) % 2],
                gsem[(w + 1) % 2])
    s_desc[AW - 1].wait()
    plsc.subcore_barrier()
    pltpu.sync_copy(acc_sh.at[pl.ds(s * RPT, RPT)], tmp_v)
    pltpu.sync_copy(tmp_v, out_hbm.at[c, pl.ds(s * RPT, RPT)])


_AGG_SCRATCH = [
    [pltpu.VMEM((AB,), jnp.int32) for _ in range(AW)],
    [pltpu.VMEM((AB,), jnp.int32) for _ in range(AW)],
    [pltpu.VMEM((AB, C), _f32) for _ in range(2)],
    pltpu.VMEM((RPT, C), _f32),
    pltpu.VMEM((RPT,), _f32),
    pltpu.VMEM((RPT,), _f32),
    pltpu.VMEM_SHARED((NPAD, C), _f32),
    pltpu.VMEM_SHARED((NPAD, C), _f32),
    [pltpu.SemaphoreType.DMA for _ in range(2)],
    [pltpu.SemaphoreType.DMA for _ in range(2)],
]


@functools.partial(
    pl.kernel,
    out_type=jax.ShapeDtypeStruct((NC, NPAD, C), _f32),
    mesh=_mesh,
    compiler_params=_sc_params,
    scratch_types=[pltpu.VMEM((RPT, C), _f32)] + _AGG_SCRATCH,
)
def _agg1(src_hbm, dst_hbm, p_hbm, degs_hbm, out_hbm,
          tab_v, sidx_v, didx_v, rows_v, tmp_v, da_v, db_v,
          ptab_sh, acc_sh, gsem, ssem):
    c = lax.axis_index("c")
    s = lax.axis_index("s")
    rbase = s * RPT

    pltpu.sync_copy(p_hbm.at[pl.ds(rbase, RPT)], tab_v)
    pltpu.sync_copy(degs_hbm.at[pl.ds(rbase, RPT)], da_v)
    pltpu.sync_copy(degs_hbm.at[pl.ds(NPAD + rbase, RPT)], db_v)

    def grp(i, carry):
        base = pl.multiple_of(i * 16, 16)
        dsum = da_v[pl.ds(base, 16)] + db_v[pl.ds(base, 16)]
        y = _rsqrt16(jnp.maximum(dsum, 1.0))
        rows = lax.broadcasted_iota(jnp.int32, (16,), 0) + base
        for j in range(C):
            cols = jnp.full((16,), j, jnp.int32)
            v = plsc.load_gather(tab_v, [rows, cols])
            plsc.store_scatter(tab_v, [rows, cols], v * y)
        return carry

    lax.fori_loop(0, RPT // 16, grp, 0)
    pltpu.sync_copy(tab_v, ptab_sh.at[pl.ds(rbase, RPT)])
    _agg_main(src_hbm, dst_hbm, out_hbm, sidx_v, didx_v, rows_v, tmp_v,
              ptab_sh, acc_sh, gsem, ssem, c, s)


@functools.partial(
    pl.kernel,
    out_type=jax.ShapeDtypeStruct((NC, NPAD, C), _f32),
    mesh=_mesh,
    compiler_params=_sc_params,
    scratch_types=[pltpu.VMEM((RPT, C), _f32), pltpu.VMEM((RPT, C), _f32),
                   pltpu.VMEM((16,), _f32),
                   pltpu.VMEM((RPT,), _f32)] + _AGG_SCRATCH,
)
def _agg2(src_hbm, dst_hbm, q_hbm, degs_hbm, degd_hbm, bc_hbm, out_hbm,
          tab_v, tab2_v, bc_v, dc_v, sidx_v, didx_v, rows_v, tmp_v,
          da_v, db_v, ptab_sh, acc_sh, gsem, ssem):
    c = lax.axis_index("c")
    s = lax.axis_index("s")
    rbase = s * RPT

    pltpu.sync_copy(q_hbm.at[0, pl.ds(rbase, RPT)], tab_v)
    pltpu.sync_copy(q_hbm.at[1, pl.ds(rbase, RPT)], tab2_v)
    pltpu.sync_copy(bc_hbm, bc_v)

    def addrow(i, carry):
        tab_v[i, :] = tab_v[i, :] + tab2_v[i, :]
        return carry

    lax.fori_loop(0, RPT, addrow, 0)

    pltpu.sync_copy(degs_hbm.at[pl.ds(rbase, RPT)], da_v)
    pltpu.sync_copy(degs_hbm.at[pl.ds(NPAD + rbase, RPT)], db_v)

    def grp_s(i, carry):
        base = pl.multiple_of(i * 16, 16)
        dsum = da_v[pl.ds(base, 16)] + db_v[pl.ds(base, 16)]
        da_v[pl.ds(base, 16)] = _rsqrt16(jnp.maximum(dsum, 1.0))
        return carry

    lax.fori_loop(0, RPT // 16, grp_s, 0)
    pltpu.sync_copy(degd_hbm.at[pl.ds(rbase, RPT)], db_v)
    pltpu.sync_copy(degd_hbm.at[pl.ds(NPAD + rbase, RPT)], dc_v)

    def grp_d(i, carry):
        base = pl.multiple_of(i * 16, 16)
        dsum = db_v[pl.ds(base, 16)] + dc_v[pl.ds(base, 16)]
        db_v[pl.ds(base, 16)] = _rsqrt16(jnp.maximum(dsum, 1.0))
        return carry

    lax.fori_loop(0, RPT // 16, grp_d, 0)

    def grp2(i, carry):
        base = pl.multiple_of(i * 16, 16)
        ds_ = da_v[pl.ds(base, 16)]
        dd_ = db_v[pl.ds(base, 16)]
        sdr = ds_ * dd_
        rows = lax.broadcasted_iota(jnp.int32, (16,), 0) + base
        for j in range(C):
            cols = jnp.full((16,), j, jnp.int32)
            bcj = plsc.load_gather(bc_v, [cols])
            v = plsc.load_gather(tab_v, [rows, cols])
            plsc.store_scatter(tab_v, [rows, cols], v * sdr + bcj * ds_)
        return carry

    lax.fori_loop(0, RPT // 16, grp2, 0)
    pltpu.sync_copy(tab_v, ptab_sh.at[pl.ds(rbase, RPT)])
    _agg_main(src_hbm, dst_hbm, out_hbm, sidx_v, didx_v, rows_v, tmp_v,
              ptab_sh, acc_sh, gsem, ssem, c, s)


def _tc_mm_body(x_ref, w1_ref, w2_ref, b1_ref, p_ref, bc_ref):
    wc = jnp.dot(w1_ref[...], w2_ref[...], preferred_element_type=_f32)
    p_ref[pl.ds(0, N), :] = jnp.dot(x_ref[...], wc,
                                    preferred_element_type=_f32)
    p_ref[pl.ds(N, NPAD - N), :] = jnp.zeros((NPAD - N, C), _f32)
    bc_ref[...] = jnp.dot(b1_ref[...], w2_ref[...],
                          preferred_element_type=_f32)


_tc_mm = pl.pallas_call(
    _tc_mm_body,
    out_shape=(
        jax.ShapeDtypeStruct((NPAD, C), _f32),
        jax.ShapeDtypeStruct((1, C), _f32),
    ),
)


def _tc_final_body(acc_ref, dd_ref, b2_ref, out_ref):
    dsum = (dd_ref[0] + dd_ref[1])[:N]
    dd = lax.rsqrt(jnp.maximum(dsum, 1.0))
    out_ref[...] = (acc_ref[0, :N, :] + acc_ref[1, :N, :]) * dd + b2_ref[...]


_tc_final = pl.pallas_call(
    _tc_final_body,
    out_shape=jax.ShapeDtypeStruct((N, C), _f32),
)


def kernel(features, edge_index, W1, b1, W2, b2):
    src = edge_index[0]
    dst = edge_index[1]

    degs_p, degd_p = _deg(src, dst)
    p_pad, bc = _tc_mm(features, W1, W2, b1.reshape(1, H))

    acc1 = _agg1(src, dst, p_pad, degs_p)
    acc2 = _agg2(src, dst, acc1, degs_p, degd_p, bc.reshape(C))
    return _tc_final(acc2, degd_p.reshape(NC, NPAD, 1), b2.reshape(1, C))


# speedup vs baseline: 24.1670x; 1.0037x over previous
import functools

import jax
import jax.numpy as jnp
from jax import lax
from jax.experimental import pallas as pl
from jax.experimental.pallas import tpu as pltpu
from jax.experimental.pallas import tpu_sc as plsc

N = 10000
E = 320000
D = 128
H = 128
C = 16

NC = 2
NS = 16
NW = NC * NS
EPT = E // NW
RPT = 640
NPAD = NS * RPT

DW, DB = 5, 2000
AW, AB = 10, 1000

_mesh = plsc.VectorSubcoreMesh(core_axis_name="c", subcore_axis_name="s")
_f32 = jnp.float32
_sc_params = pltpu.CompilerParams(use_tc_tiling_on_sc=False, needs_layout_passes=False)


def _rsqrt16(x):
    bits = plsc.bitcast(x, jnp.int32)
    y = plsc.bitcast(jnp.int32(0x5F3759DF) - (bits >> 1), _f32)
    for _ in range(3):
        y = y * (1.5 - 0.5 * x * y * y)
    return y


@functools.partial(
    pl.kernel,
    out_type=(
        jax.ShapeDtypeStruct((NW * RPT,), _f32),
        jax.ShapeDtypeStruct((NW * RPT,), _f32),
    ),
    mesh=_mesh,
    scratch_types=[
        [pltpu.VMEM((DB,), jnp.int32) for _ in range(DW)],
        [pltpu.VMEM((DB,), jnp.int32) for _ in range(DW)],
        pltpu.VMEM((DB,), _f32),
        pltpu.VMEM((RPT,), _f32),
        pltpu.VMEM_SHARED((NPAD,), _f32),
        pltpu.VMEM_SHARED((NPAD,), _f32),
    ],
)
def _deg(src_hbm, dst_hbm, outs_hbm, outd_hbm,
         sidx_v, didx_v, ones_v, tmp_v, degs_sh, degd_sh):
    c = lax.axis_index("c")
    s = lax.axis_index("s")
    wid = c * NS + s
    ebase = wid * EPT

    def frow(i, carry):
        ones_v[pl.ds(pl.multiple_of(i * 16, 16), 16)] = jnp.ones((16,), _f32)
        return carry

    lax.fori_loop(0, DB // 16, frow, 0)

    def zrow(i, carry):
        tmp_v[pl.ds(pl.multiple_of(i * 16, 16), 16)] = jnp.zeros((16,), _f32)
        return carry

    lax.fori_loop(0, RPT // 16, zrow, 0)
    pltpu.sync_copy(tmp_v, degs_sh.at[pl.ds(s * RPT, RPT)])
    pltpu.sync_copy(tmp_v, degd_sh.at[pl.ds(s * RPT, RPT)])
    for w in range(DW):
        pltpu.sync_copy(src_hbm.at[pl.ds(ebase + w * DB, DB)], sidx_v[w])
        pltpu.sync_copy(dst_hbm.at[pl.ds(ebase + w * DB, DB)], didx_v[w])
    plsc.subcore_barrier()
    for w in range(DW):
        pltpu.sync_copy(ones_v, degs_sh.at[sidx_v[w]], add=True)
        pltpu.sync_copy(ones_v, degd_sh.at[didx_v[w]], add=True)
    plsc.subcore_barrier()
    pltpu.sync_copy(degs_sh.at[pl.ds(s * RPT, RPT)], tmp_v)
    pltpu.sync_copy(tmp_v, outs_hbm.at[pl.ds(wid * RPT, RPT)])
    pltpu.sync_copy(degd_sh.at[pl.ds(s * RPT, RPT)], tmp_v)
    pltpu.sync_copy(tmp_v, outd_hbm.at[pl.ds(wid * RPT, RPT)])


def _agg_main(src_hbm, dst_hbm, out_hbm, sidx_v, didx_v, rows_v, tmp_v,
              ptab_sh, acc_sh, gsem, ssem, c, s):
    wid = c * NS + s
    ebase = wid * EPT

    def zrow(i, carry):
        tmp_v[i, :] = jnp.zeros((C,), _f32)
        return carry

    lax.fori_loop(0, RPT, zrow, 0)
    pltpu.sync_copy(tmp_v, acc_sh.at[pl.ds(s * RPT, RPT)])
    for w in range(AW):
        pltpu.sync_copy(src_hbm.at[pl.ds(ebase + w * AB, AB)], sidx_v[w])
        pltpu.sync_copy(dst_hbm.at[pl.ds(ebase + w * AB, AB)], didx_v[w])
    plsc.subcore_barrier()

    g_desc = [None] * AW
    s_desc = [None] * AW
    g_desc[0] = pltpu.async_copy(ptab_sh.at[sidx_v[0]], rows_v[0], gsem[0])
    for w in range(AW):
        g_desc[w].wait()
        s_desc[w] = pltpu.async_copy(
            rows_v[w % 2], acc_sh.at[didx_v[w]], ssem[w % 2], add=True)
        if w + 1 < AW:
            if w >= 1:
                s_desc[w - 1].wait()
            g_desc[w + 1] = pltpu.async_copy(
                ptab_sh.at[sidx_v[w + 1]], rows_v[(w + 1) % 2],
                gsem[(w + 1) % 2])
    s_desc[AW - 1].wait()
    plsc.subcore_barrier()
    pltpu.sync_copy(acc_sh.at[pl.ds(s * RPT, RPT)], tmp_v)
    pltpu.sync_copy(tmp_v, out_hbm.at[c, pl.ds(s * RPT, RPT)])


_AGG_SCRATCH = [
    [pltpu.VMEM((AB,), jnp.int32) for _ in range(AW)],
    [pltpu.VMEM((AB,), jnp.int32) for _ in range(AW)],
    [pltpu.VMEM((AB, C), _f32) for _ in range(2)],
    pltpu.VMEM((RPT, C), _f32),
    pltpu.VMEM((RPT,), _f32),
    pltpu.VMEM((RPT,), _f32),
    pltpu.VMEM_SHARED((NPAD, C), _f32),
    pltpu.VMEM_SHARED((NPAD, C), _f32),
    [pltpu.SemaphoreType.DMA for _ in range(2)],
    [pltpu.SemaphoreType.DMA for _ in range(2)],
]


@functools.partial(
    pl.kernel,
    out_type=jax.ShapeDtypeStruct((NC, NPAD, C), _f32),
    mesh=_mesh,
    compiler_params=_sc_params,
    scratch_types=[pltpu.VMEM((RPT, C), _f32)] + _AGG_SCRATCH,
)
def _agg1(src_hbm, dst_hbm, p_hbm, degs_hbm, out_hbm,
          tab_v, sidx_v, didx_v, rows_v, tmp_v, da_v, db_v,
          ptab_sh, acc_sh, gsem, ssem):
    c = lax.axis_index("c")
    s = lax.axis_index("s")
    rbase = s * RPT

    pltpu.sync_copy(p_hbm.at[pl.ds(rbase, RPT)], tab_v)
    pltpu.sync_copy(degs_hbm.at[pl.ds(rbase, RPT)], da_v)
    pltpu.sync_copy(degs_hbm.at[pl.ds(NPAD + rbase, RPT)], db_v)

    def grp(i, carry):
        base = pl.multiple_of(i * 16, 16)
        dsum = da_v[pl.ds(base, 16)] + db_v[pl.ds(base, 16)]
        y = _rsqrt16(jnp.maximum(dsum, 1.0))
        rows = lax.broadcasted_iota(jnp.int32, (16,), 0) + base
        for j in range(C):
            cols = jnp.full((16,), j, jnp.int32)
            v = plsc.load_gather(tab_v, [rows, cols])
            plsc.store_scatter(tab_v, [rows, cols], v * y)
        return carry

    lax.fori_loop(0, RPT // 16, grp, 0)
    pltpu.sync_copy(tab_v, ptab_sh.at[pl.ds(rbase, RPT)])
    _agg_main(src_hbm, dst_hbm, out_hbm, sidx_v, didx_v, rows_v, tmp_v,
              ptab_sh, acc_sh, gsem, ssem, c, s)


@functools.partial(
    pl.kernel,
    out_type=jax.ShapeDtypeStruct((NC, NPAD, C), _f32),
    mesh=_mesh,
    compiler_params=_sc_params,
    scratch_types=[pltpu.VMEM((RPT, C), _f32), pltpu.VMEM((RPT, C), _f32),
                   pltpu.VMEM((16,), _f32),
                   pltpu.VMEM((RPT,), _f32)] + _AGG_SCRATCH,
)
def _agg2(src_hbm, dst_hbm, q_hbm, degs_hbm, degd_hbm, bc_hbm, out_hbm,
          tab_v, tab2_v, bc_v, dc_v, sidx_v, didx_v, rows_v, tmp_v,
          da_v, db_v, ptab_sh, acc_sh, gsem, ssem):
    c = lax.axis_index("c")
    s = lax.axis_index("s")
    rbase = s * RPT

    pltpu.sync_copy(q_hbm.at[0, pl.ds(rbase, RPT)], tab_v)
    pltpu.sync_copy(q_hbm.at[1, pl.ds(rbase, RPT)], tab2_v)
    pltpu.sync_copy(bc_hbm, bc_v)

    def addrow(i, carry):
        tab_v[i, :] = tab_v[i, :] + tab2_v[i, :]
        return carry

    lax.fori_loop(0, RPT, addrow, 0)

    pltpu.sync_copy(degs_hbm.at[pl.ds(rbase, RPT)], da_v)
    pltpu.sync_copy(degs_hbm.at[pl.ds(NPAD + rbase, RPT)], db_v)

    def grp_s(i, carry):
        base = pl.multiple_of(i * 16, 16)
        dsum = da_v[pl.ds(base, 16)] + db_v[pl.ds(base, 16)]
        da_v[pl.ds(base, 16)] = _rsqrt16(jnp.maximum(dsum, 1.0))
        return carry

    lax.fori_loop(0, RPT // 16, grp_s, 0)
    pltpu.sync_copy(degd_hbm.at[pl.ds(rbase, RPT)], db_v)
    pltpu.sync_copy(degd_hbm.at[pl.ds(NPAD + rbase, RPT)], dc_v)

    def grp_d(i, carry):
        base = pl.multiple_of(i * 16, 16)
        dsum = db_v[pl.ds(base, 16)] + dc_v[pl.ds(base, 16)]
        db_v[pl.ds(base, 16)] = _rsqrt16(jnp.maximum(dsum, 1.0))
        return carry

    lax.fori_loop(0, RPT // 16, grp_d, 0)

    def grp2(i, carry):
        base = pl.multiple_of(i * 16, 16)
        ds_ = da_v[pl.ds(base, 16)]
        dd_ = db_v[pl.ds(base, 16)]
        sdr = ds_ * dd_
        rows = lax.broadcasted_iota(jnp.int32, (16,), 0) + base
        for j in range(C):
            cols = jnp.full((16,), j, jnp.int32)
            bcj = plsc.load_gather(bc_v, [cols])
            v = plsc.load_gather(tab_v, [rows, cols])
            plsc.store_scatter(tab_v, [rows, cols], v * sdr + bcj * ds_)
        return carry

    lax.fori_loop(0, RPT // 16, grp2, 0)
    pltpu.sync_copy(tab_v, ptab_sh.at[pl.ds(rbase, RPT)])
    _agg_main(src_hbm, dst_hbm, out_hbm, sidx_v, didx_v, rows_v, tmp_v,
              ptab_sh, acc_sh, gsem, ssem, c, s)


def _tc_mm_body(x_ref, w1_ref, w2_ref, b1_ref, p_ref, bc_ref):
    wc = jnp.dot(w1_ref[...], w2_ref[...], preferred_element_type=_f32)
    p_ref[...] = jnp.dot(x_ref[...], wc, preferred_element_type=_f32)
    bc_ref[...] = jnp.dot(b1_ref[...], w2_ref[...],
                          preferred_element_type=_f32)


_MMB = 2000


_tc_mm = pl.pallas_call(
    _tc_mm_body,
    grid=(N // _MMB,),
    in_specs=[
        pl.BlockSpec((_MMB, D), lambda i: (i, 0)),
        pl.BlockSpec((D, H), lambda i: (0, 0)),
        pl.BlockSpec((H, C), lambda i: (0, 0)),
        pl.BlockSpec((1, H), lambda i: (0, 0)),
    ],
    out_specs=(
        pl.BlockSpec((_MMB, C), lambda i: (i, 0)),
        pl.BlockSpec((1, C), lambda i: (0, 0)),
    ),
    out_shape=(
        jax.ShapeDtypeStruct((NPAD, C), _f32),
        jax.ShapeDtypeStruct((1, C), _f32),
    ),
)


def _tc_final_body(acc_ref, dd_ref, b2_ref, out_ref):
    dsum = (dd_ref[0] + dd_ref[1])[:N]
    dd = lax.rsqrt(jnp.maximum(dsum, 1.0))
    out_ref[...] = (acc_ref[0, :N, :] + acc_ref[1, :N, :]) * dd + b2_ref[...]


_tc_final = pl.pallas_call(
    _tc_final_body,
    out_shape=jax.ShapeDtypeStruct((N, C), _f32),
)


def kernel(features, edge_index, W1, b1, W2, b2):
    src = edge_index[0]
    dst = edge_index[1]

    degs_p, degd_p = _deg(src, dst)
    p_pad, bc = _tc_mm(features, W1, W2, b1.reshape(1, H))

    acc1 = _agg1(src, dst, p_pad, degs_p)
    acc2 = _agg2(src, dst, acc1, degs_p, degd_p, bc.reshape(C))
    return _tc_final(acc2, degd_p.reshape(NC, NPAD, 1), b2.reshape(1, C))


# speedup vs baseline: 24.3162x; 1.0062x over previous
import functools

import jax
import jax.numpy as jnp
from jax import lax
from jax.experimental import pallas as pl
from jax.experimental.pallas import tpu as pltpu
from jax.experimental.pallas import tpu_sc as plsc

N = 10000
E = 320000
D = 128
H = 128
C = 16

NC = 2
NS = 16
NW = NC * NS
EPT = E // NW
RPT = 640
NPAD = NS * RPT

DW, DB = 5, 2000
AW, AB = 10, 1000

_mesh = plsc.VectorSubcoreMesh(core_axis_name="c", subcore_axis_name="s")
_f32 = jnp.float32
_sc_params = pltpu.CompilerParams(use_tc_tiling_on_sc=False, needs_layout_passes=False)


def _rsqrt16(x):
    bits = plsc.bitcast(x, jnp.int32)
    y = plsc.bitcast(jnp.int32(0x5F3759DF) - (bits >> 1), _f32)
    for _ in range(3):
        y = y * (1.5 - 0.5 * x * y * y)
    return y


@functools.partial(
    pl.kernel,
    out_type=(
        jax.ShapeDtypeStruct((NW * RPT,), _f32),
        jax.ShapeDtypeStruct((NW * RPT,), _f32),
    ),
    mesh=_mesh,
    scratch_types=[
        [pltpu.VMEM((DB,), jnp.int32) for _ in range(DW)],
        [pltpu.VMEM((DB,), jnp.int32) for _ in range(DW)],
        pltpu.VMEM((DB,), _f32),
        pltpu.VMEM((RPT,), _f32),
        pltpu.VMEM_SHARED((NPAD,), _f32),
        pltpu.VMEM_SHARED((NPAD,), _f32),
    ],
)
def _deg(src_hbm, dst_hbm, outs_hbm, outd_hbm,
         sidx_v, didx_v, ones_v, tmp_v, degs_sh, degd_sh):
    c = lax.axis_index("c")
    s = lax.axis_index("s")
    wid = c * NS + s
    ebase = wid * EPT

    def frow(i, carry):
        ones_v[pl.ds(pl.multiple_of(i * 16, 16), 16)] = jnp.ones((16,), _f32)
        return carry

    lax.fori_loop(0, DB // 16, frow, 0)

    def zrow(i, carry):
        tmp_v[pl.ds(pl.multiple_of(i * 16, 16), 16)] = jnp.zeros((16,), _f32)
        return carry

    lax.fori_loop(0, RPT // 16, zrow, 0)
    pltpu.sync_copy(tmp_v, degs_sh.at[pl.ds(s * RPT, RPT)])
    pltpu.sync_copy(tmp_v, degd_sh.at[pl.ds(s * RPT, RPT)])
    for w in range(DW):
        pltpu.sync_copy(src_hbm.at[pl.ds(ebase + w * DB, DB)], sidx_v[w])
        pltpu.sync_copy(dst_hbm.at[pl.ds(ebase + w * DB, DB)], didx_v[w])
    plsc.subcore_barrier()
    for w in range(DW):
        pltpu.sync_copy(ones_v, degs_sh.at[sidx_v[w]], add=True)
        pltpu.sync_copy(ones_v, degd_sh.at[didx_v[w]], add=True)
    plsc.subcore_barrier()
    pltpu.sync_copy(degs_sh.at[pl.ds(s * RPT, RPT)], tmp_v)
    pltpu.sync_copy(tmp_v, outs_hbm.at[pl.ds(wid * RPT, RPT)])
    pltpu.sync_copy(degd_sh.at[pl.ds(s * RPT, RPT)], tmp_v)
    pltpu.sync_copy(tmp_v, outd_hbm.at[pl.ds(wid * RPT, RPT)])


def _agg_main(src_hbm, dst_hbm, out_hbm, sidx_v, didx_v, rows_v, tmp_v,
              ptab_sh, acc_sh, gsem, ssem, c, s):
    wid = c * NS + s
    ebase = wid * EPT

    def zrow(i, carry):
        tmp_v[i, :] = jnp.zeros((C,), _f32)
        return carry

    lax.fori_loop(0, RPT, zrow, 0)
    pltpu.sync_copy(tmp_v, acc_sh.at[pl.ds(s * RPT, RPT)])
    for w in range(AW):
        pltpu.sync_copy(src_hbm.at[pl.ds(ebase + w * AB, AB)], sidx_v[w])
        pltpu.sync_copy(dst_hbm.at[pl.ds(ebase + w * AB, AB)], didx_v[w])
    plsc.subcore_barrier()

    g_desc = [None] * AW
    s_desc = [None] * AW
    g_desc[0] = pltpu.async_copy(ptab_sh.at[sidx_v[0]], rows_v[0], gsem[0])
    for w in range(AW):
        g_desc[w].wait()
        s_desc[w] = pltpu.async_copy(
            rows_v[w % 2], acc_sh.at[didx_v[w]], ssem[w % 2], add=True)
        if w + 1 < AW:
            if w >= 1:
                s_desc[w - 1].wait()
            g_desc[w + 1] = pltpu.async_copy(
                ptab_sh.at[sidx_v[w + 1]], rows_v[(w + 1) % 2],
                gsem[(w + 1) % 2])
    s_desc[AW - 1].wait()
    plsc.subcore_barrier()
    pltpu.sync_copy(acc_sh.at[pl.ds(s * RPT, RPT)], tmp_v)
    pltpu.sync_copy(tmp_v, out_hbm.at[c, pl.ds(s * RPT, RPT)])


_AGG_SCRATCH = [
    [pltpu.VMEM((AB,), jnp.int32) for _ in range(AW)],
    [pltpu.VMEM((AB,), jnp.int32) for _ in range(AW)],
    [pltpu.VMEM((AB, C), _f32) for _ in range(2)],
    pltpu.VMEM((RPT, C), _f32),
    pltpu.VMEM((RPT,), _f32),
    pltpu.VMEM((RPT,), _f32),
    pltpu.VMEM_SHARED((NPAD, C), _f32),
    pltpu.VMEM_SHARED((NPAD, C), _f32),
    [pltpu.SemaphoreType.DMA for _ in range(2)],
    [pltpu.SemaphoreType.DMA for _ in range(2)],
]


@functools.partial(
    pl.kernel,
    out_type=jax.ShapeDtypeStruct((NC, NPAD, C), _f32),
    mesh=_mesh,
    compiler_params=_sc_params,
    scratch_types=[pltpu.VMEM((RPT, C), _f32)] + _AGG_SCRATCH,
)
def _agg1(src_hbm, dst_hbm, p_hbm, degs_hbm, out_hbm,
          tab_v, sidx_v, didx_v, rows_v, tmp_v, da_v, db_v,
          ptab_sh, acc_sh, gsem, ssem):
    c = lax.axis_index("c")
    s = lax.axis_index("s")
    rbase = s * RPT

    pltpu.sync_copy(p_hbm.at[pl.ds(rbase, RPT)], tab_v)
    pltpu.sync_copy(degs_hbm.at[pl.ds(rbase, RPT)], da_v)
    pltpu.sync_copy(degs_hbm.at[pl.ds(NPAD + rbase, RPT)], db_v)

    def grp(i, carry):
        base = pl.multiple_of(i * 16, 16)
        dsum = da_v[pl.ds(base, 16)] + db_v[pl.ds(base, 16)]
        y = _rsqrt16(jnp.maximum(dsum, 1.0))
        rows = lax.broadcasted_iota(jnp.int32, (16,), 0) + base
        for j in range(C):
            cols = jnp.full((16,), j, jnp.int32)
            v = plsc.load_gather(tab_v, [rows, cols])
            plsc.store_scatter(tab_v, [rows, cols], v * y)
        return carry

    lax.fori_loop(0, RPT // 16, grp, 0)
    pltpu.sync_copy(tab_v, ptab_sh.at[pl.ds(rbase, RPT)])
    _agg_main(src_hbm, dst_hbm, out_hbm, sidx_v, didx_v, rows_v, tmp_v,
              ptab_sh, acc_sh, gsem, ssem, c, s)


@functools.partial(
    pl.kernel,
    out_type=jax.ShapeDtypeStruct((NC, NPAD, C), _f32),
    mesh=_mesh,
    compiler_params=_sc_params,
    scratch_types=[pltpu.VMEM((RPT, C), _f32), pltpu.VMEM((RPT, C), _f32),
                   pltpu.VMEM((16,), _f32),
                   pltpu.VMEM((RPT,), _f32)] + _AGG_SCRATCH,
)
def _agg2(src_hbm, dst_hbm, q_hbm, degs_hbm, degd_hbm, bc_hbm, out_hbm,
          tab_v, tab2_v, bc_v, dc_v, sidx_v, didx_v, rows_v, tmp_v,
          da_v, db_v, ptab_sh, acc_sh, gsem, ssem):
    c = lax.axis_index("c")
    s = lax.axis_index("s")
    rbase = s * RPT

    pltpu.sync_copy(q_hbm.at[0, pl.ds(rbase, RPT)], tab_v)
    pltpu.sync_copy(q_hbm.at[1, pl.ds(rbase, RPT)], tab2_v)
    pltpu.sync_copy(bc_hbm, bc_v)

    def addrow(i, carry):
        tab_v[i, :] = tab_v[i, :] + tab2_v[i, :]
        return carry

    lax.fori_loop(0, RPT, addrow, 0)

    pltpu.sync_copy(degs_hbm.at[pl.ds(rbase, RPT)], da_v)
    pltpu.sync_copy(degs_hbm.at[pl.ds(NPAD + rbase, RPT)], db_v)

    def grp_s(i, carry):
        base = pl.multiple_of(i * 16, 16)
        dsum = da_v[pl.ds(base, 16)] + db_v[pl.ds(base, 16)]
        da_v[pl.ds(base, 16)] = _rsqrt16(jnp.maximum(dsum, 1.0))
        return carry

    lax.fori_loop(0, RPT // 16, grp_s, 0)
    pltpu.sync_copy(degd_hbm.at[pl.ds(rbase, RPT)], db_v)
    pltpu.sync_copy(degd_hbm.at[pl.ds(NPAD + rbase, RPT)], dc_v)

    def grp_d(i, carry):
        base = pl.multiple_of(i * 16, 16)
        dsum = db_v[pl.ds(base, 16)] + dc_v[pl.ds(base, 16)]
        db_v[pl.ds(base, 16)] = _rsqrt16(jnp.maximum(dsum, 1.0))
        return carry

    lax.fori_loop(0, RPT // 16, grp_d, 0)

    def grp2(i, carry):
        base = pl.multiple_of(i * 16, 16)
        ds_ = da_v[pl.ds(base, 16)]
        dd_ = db_v[pl.ds(base, 16)]
        sdr = ds_ * dd_
        rows = lax.broadcasted_iota(jnp.int32, (16,), 0) + base
        for j in range(C):
            cols = jnp.full((16,), j, jnp.int32)
            bcj = plsc.load_gather(bc_v, [cols])
            v = plsc.load_gather(tab_v, [rows, cols])
            plsc.store_scatter(tab_v, [rows, cols], v * sdr + bcj * ds_)
        return carry

    lax.fori_loop(0, RPT // 16, grp2, 0)
    pltpu.sync_copy(tab_v, ptab_sh.at[pl.ds(rbase, RPT)])
    _agg_main(src_hbm, dst_hbm, out_hbm, sidx_v, didx_v, rows_v, tmp_v,
              ptab_sh, acc_sh, gsem, ssem, c, s)


def _tc_mm_body(x_ref, w1_ref, w2_ref, b1_ref, p_ref, bc_ref):
    wc = jnp.dot(w1_ref[...], w2_ref[...], preferred_element_type=_f32)
    p_ref[...] = jnp.dot(x_ref[...], wc, preferred_element_type=_f32)
    bc_ref[...] = jnp.dot(b1_ref[...], w2_ref[...],
                          preferred_element_type=_f32)


_MMB = 2000


_tc_mm = pl.pallas_call(
    _tc_mm_body,
    grid=(N // _MMB,),
    in_specs=[
        pl.BlockSpec((_MMB, D), lambda i: (i, 0)),
        pl.BlockSpec((D, H), lambda i: (0, 0)),
        pl.BlockSpec((H, C), lambda i: (0, 0)),
        pl.BlockSpec((1, H), lambda i: (0, 0)),
    ],
    out_specs=(
        pl.BlockSpec((_MMB, C), lambda i: (i, 0)),
        pl.BlockSpec((1, C), lambda i: (0, 0)),
    ),
    out_shape=(
        jax.ShapeDtypeStruct((NPAD, C), _f32),
        jax.ShapeDtypeStruct((1, C), _f32),
    ),
)


RPF = NPAD // NW


@functools.partial(
    pl.kernel,
    out_type=jax.ShapeDtypeStruct((NPAD, C), _f32),
    mesh=_mesh,
    compiler_params=_sc_params,
    scratch_types=[
        pltpu.VMEM((RPF, C), _f32),
        pltpu.VMEM((RPF, C), _f32),
        pltpu.VMEM((RPF,), _f32),
        pltpu.VMEM((RPF,), _f32),
        pltpu.VMEM((16,), _f32),
    ],
)
def _fin(acc_hbm, degd_hbm, b2_hbm, out_hbm, tab_v, tab2_v, da_v, db_v, b2_v):
    c = lax.axis_index("c")
    s = lax.axis_index("s")
    wid = c * NS + s
    rbase = wid * RPF

    pltpu.sync_copy(acc_hbm.at[0, pl.ds(rbase, RPF)], tab_v)
    pltpu.sync_copy(acc_hbm.at[1, pl.ds(rbase, RPF)], tab2_v)
    pltpu.sync_copy(degd_hbm.at[pl.ds(rbase, RPF)], da_v)
    pltpu.sync_copy(degd_hbm.at[pl.ds(NPAD + rbase, RPF)], db_v)
    pltpu.sync_copy(b2_hbm, b2_v)

    def addrow(i, carry):
        tab_v[i, :] = tab_v[i, :] + tab2_v[i, :]
        return carry

    lax.fori_loop(0, RPF, addrow, 0)

    def grp_d(i, carry):
        base = pl.multiple_of(i * 16, 16)
        dsum = da_v[pl.ds(base, 16)] + db_v[pl.ds(base, 16)]
        da_v[pl.ds(base, 16)] = _rsqrt16(jnp.maximum(dsum, 1.0))
        return carry

    lax.fori_loop(0, RPF // 16, grp_d, 0)

    def grp(i, carry):
        base = pl.multiple_of(i * 16, 16)
        dd_ = da_v[pl.ds(base, 16)]
        rows = lax.broadcasted_iota(jnp.int32, (16,), 0) + base
        for j in range(C):
            cols = jnp.full((16,), j, jnp.int32)
            b2j = plsc.load_gather(b2_v, [cols])
            v = plsc.load_gather(tab_v, [rows, cols])
            plsc.store_scatter(tab_v, [rows, cols], v * dd_ + b2j)
        return carry

    lax.fori_loop(0, RPF // 16, grp, 0)
    pltpu.sync_copy(tab_v, out_hbm.at[pl.ds(rbase, RPF)])


def kernel(features, edge_index, W1, b1, W2, b2):
    src = edge_index[0]
    dst = edge_index[1]

    degs_p, degd_p = _deg(src, dst)
    p_pad, bc = _tc_mm(features, W1, W2, b1.reshape(1, H))

    acc1 = _agg1(src, dst, p_pad, degs_p)
    acc2 = _agg2(src, dst, acc1, degs_p, degd_p, bc.reshape(C))
    return _fin(acc2, degd_p, b2)[:N]


# speedup vs baseline: 25.9404x; 1.0668x over previous
import functools

import jax
import jax.numpy as jnp
from jax import lax
from jax.experimental import pallas as pl
from jax.experimental.pallas import tpu as pltpu
from jax.experimental.pallas import tpu_sc as plsc

N = 10000
E = 320000
D = 128
H = 128
C = 16

NC = 2
NS = 16
NW = NC * NS
EPT = E // NW
RPT = 640
NPAD = NS * RPT

DW, DB = 5, 2000
AW, AB = 10, 1000

_mesh = plsc.VectorSubcoreMesh(core_axis_name="c", subcore_axis_name="s")
_f32 = jnp.float32
_sc_params = pltpu.CompilerParams(use_tc_tiling_on_sc=False, needs_layout_passes=False)


def _rsqrt16(x):
    bits = plsc.bitcast(x, jnp.int32)
    y = plsc.bitcast(jnp.int32(0x5F3759DF) - (bits >> 1), _f32)
    for _ in range(3):
        y = y * (1.5 - 0.5 * x * y * y)
    return y


@functools.partial(
    pl.kernel,
    out_type=(
        jax.ShapeDtypeStruct((NW * RPT,), _f32),
        jax.ShapeDtypeStruct((NW * RPT,), _f32),
    ),
    mesh=_mesh,
    compiler_params=_sc_params,
    scratch_types=[
        [pltpu.VMEM((DB,), jnp.int32) for _ in range(DW)],
        [pltpu.VMEM((DB,), jnp.int32) for _ in range(DW)],
        pltpu.VMEM((DB,), _f32),
        pltpu.VMEM((RPT,), _f32),
        pltpu.VMEM_SHARED((NPAD,), _f32),
        pltpu.VMEM_SHARED((NPAD,), _f32),
    ],
)
def _deg(ei_hbm, outs_hbm, outd_hbm,
         sidx_v, didx_v, ones_v, tmp_v, degs_sh, degd_sh):
    c = lax.axis_index("c")
    s = lax.axis_index("s")
    wid = c * NS + s
    ebase = wid * EPT

    def frow(i, carry):
        ones_v[pl.ds(pl.multiple_of(i * 16, 16), 16)] = jnp.ones((16,), _f32)
        return carry

    lax.fori_loop(0, DB // 16, frow, 0)

    def zrow(i, carry):
        tmp_v[pl.ds(pl.multiple_of(i * 16, 16), 16)] = jnp.zeros((16,), _f32)
        return carry

    lax.fori_loop(0, RPT // 16, zrow, 0)
    pltpu.sync_copy(tmp_v, degs_sh.at[pl.ds(s * RPT, RPT)])
    pltpu.sync_copy(tmp_v, degd_sh.at[pl.ds(s * RPT, RPT)])
    for w in range(DW):
        pltpu.sync_copy(ei_hbm.at[0, pl.ds(ebase + w * DB, DB)], sidx_v[w])
        pltpu.sync_copy(ei_hbm.at[1, pl.ds(ebase + w * DB, DB)], didx_v[w])
    plsc.subcore_barrier()
    for w in range(DW):
        pltpu.sync_copy(ones_v, degs_sh.at[sidx_v[w]], add=True)
        pltpu.sync_copy(ones_v, degd_sh.at[didx_v[w]], add=True)
    plsc.subcore_barrier()
    pltpu.sync_copy(degs_sh.at[pl.ds(s * RPT, RPT)], tmp_v)
    pltpu.sync_copy(tmp_v, outs_hbm.at[pl.ds(wid * RPT, RPT)])
    pltpu.sync_copy(degd_sh.at[pl.ds(s * RPT, RPT)], tmp_v)
    pltpu.sync_copy(tmp_v, outd_hbm.at[pl.ds(wid * RPT, RPT)])


def _agg_main(ei_hbm, out_hbm, sidx_v, didx_v, rows_v, tmp_v,
              ptab_sh, acc_sh, gsem, ssem, c, s):
    wid = c * NS + s
    ebase = wid * EPT

    def zrow(i, carry):
        tmp_v[i, :] = jnp.zeros((C,), _f32)
        return carry

    lax.fori_loop(0, RPT, zrow, 0)
    pltpu.sync_copy(tmp_v, acc_sh.at[pl.ds(s * RPT, RPT)])
    for w in range(AW):
        pltpu.sync_copy(ei_hbm.at[0, pl.ds(ebase + w * AB, AB)], sidx_v[w])
        pltpu.sync_copy(ei_hbm.at[1, pl.ds(ebase + w * AB, AB)], didx_v[w])
    plsc.subcore_barrier()

    g_desc = [None] * AW
    s_desc = [None] * AW
    g_desc[0] = pltpu.async_copy(ptab_sh.at[sidx_v[0]], rows_v[0], gsem[0])
    for w in range(AW):
        g_desc[w].wait()
        s_desc[w] = pltpu.async_copy(
            rows_v[w % 2], acc_sh.at[didx_v[w]], ssem[w % 2], add=True)
        if w + 1 < AW:
            if w >= 1:
                s_desc[w - 1].wait()
            g_desc[w + 1] = pltpu.async_copy(
                ptab_sh.at[sidx_v[w + 1]], rows_v[(w + 1) % 2],
                gsem[(w + 1) % 2])
    s_desc[AW - 1].wait()
    plsc.subcore_barrier()
    pltpu.sync_copy(acc_sh.at[pl.ds(s * RPT, RPT)], tmp_v)
    pltpu.sync_copy(tmp_v, out_hbm.at[c, pl.ds(s * RPT, RPT)])


_AGG_SCRATCH = [
    [pltpu.VMEM((AB,), jnp.int32) for _ in range(AW)],
    [pltpu.VMEM((AB,), jnp.int32) for _ in range(AW)],
    [pltpu.VMEM((AB, C), _f32) for _ in range(2)],
    pltpu.VMEM((RPT, C), _f32),
    pltpu.VMEM((RPT,), _f32),
    pltpu.VMEM((RPT,), _f32),
    pltpu.VMEM_SHARED((NPAD, C), _f32),
    pltpu.VMEM_SHARED((NPAD, C), _f32),
    [pltpu.SemaphoreType.DMA for _ in range(2)],
    [pltpu.SemaphoreType.DMA for _ in range(2)],
]


@functools.partial(
    pl.kernel,
    out_type=jax.ShapeDtypeStruct((NC, NPAD, C), _f32),
    mesh=_mesh,
    compiler_params=_sc_params,
    scratch_types=[pltpu.VMEM((RPT, C), _f32)] + _AGG_SCRATCH,
)
def _agg1(ei_hbm, p_hbm, degs_hbm, out_hbm,
          tab_v, sidx_v, didx_v, rows_v, tmp_v, da_v, db_v,
          ptab_sh, acc_sh, gsem, ssem):
    c = lax.axis_index("c")
    s = lax.axis_index("s")
    rbase = s * RPT

    pltpu.sync_copy(p_hbm.at[pl.ds(rbase, RPT)], tab_v)
    pltpu.sync_copy(degs_hbm.at[pl.ds(rbase, RPT)], da_v)
    pltpu.sync_copy(degs_hbm.at[pl.ds(NPAD + rbase, RPT)], db_v)

    def grp(i, carry):
        base = pl.multiple_of(i * 16, 16)
        dsum = da_v[pl.ds(base, 16)] + db_v[pl.ds(base, 16)]
        y = _rsqrt16(jnp.maximum(dsum, 1.0))
        rows = lax.broadcasted_iota(jnp.int32, (16,), 0) + base
        for j in range(C):
            cols = jnp.full((16,), j, jnp.int32)
            v = plsc.load_gather(tab_v, [rows, cols])
            plsc.store_scatter(tab_v, [rows, cols], v * y)
        return carry

    lax.fori_loop(0, RPT // 16, grp, 0)
    pltpu.sync_copy(tab_v, ptab_sh.at[pl.ds(rbase, RPT)])
    _agg_main(ei_hbm, out_hbm, sidx_v, didx_v, rows_v, tmp_v,
              ptab_sh, acc_sh, gsem, ssem, c, s)


@functools.partial(
    pl.kernel,
    out_type=jax.ShapeDtypeStruct((NC, NPAD, C), _f32),
    mesh=_mesh,
    compiler_params=_sc_params,
    scratch_types=[pltpu.VMEM((RPT, C), _f32), pltpu.VMEM((RPT, C), _f32),
                   pltpu.VMEM((16,), _f32),
                   pltpu.VMEM((RPT,), _f32)] + _AGG_SCRATCH,
)
def _agg2(ei_hbm, q_hbm, degs_hbm, degd_hbm, bc_hbm, out_hbm,
          tab_v, tab2_v, bc_v, dc_v, sidx_v, didx_v, rows_v, tmp_v,
          da_v, db_v, ptab_sh, acc_sh, gsem, ssem):
    c = lax.axis_index("c")
    s = lax.axis_index("s")
    rbase = s * RPT

    pltpu.sync_copy(q_hbm.at[0, pl.ds(rbase, RPT)], tab_v)
    pltpu.sync_copy(q_hbm.at[1, pl.ds(rbase, RPT)], tab2_v)
    pltpu.sync_copy(bc_hbm, bc_v)

    def addrow(i, carry):
        tab_v[i, :] = tab_v[i, :] + tab2_v[i, :]
        return carry

    lax.fori_loop(0, RPT, addrow, 0)

    pltpu.sync_copy(degs_hbm.at[pl.ds(rbase, RPT)], da_v)
    pltpu.sync_copy(degs_hbm.at[pl.ds(NPAD + rbase, RPT)], db_v)

    def grp_s(i, carry):
        base = pl.multiple_of(i * 16, 16)
        dsum = da_v[pl.ds(base, 16)] + db_v[pl.ds(base, 16)]
        da_v[pl.ds(base, 16)] = _rsqrt16(jnp.maximum(dsum, 1.0))
        return carry

    lax.fori_loop(0, RPT // 16, grp_s, 0)
    pltpu.sync_copy(degd_hbm.at[pl.ds(rbase, RPT)], db_v)
    pltpu.sync_copy(degd_hbm.at[pl.ds(NPAD + rbase, RPT)], dc_v)

    def grp_d(i, carry):
        base = pl.multiple_of(i * 16, 16)
        dsum = db_v[pl.ds(base, 16)] + dc_v[pl.ds(base, 16)]
        db_v[pl.ds(base, 16)] = _rsqrt16(jnp.maximum(dsum, 1.0))
        return carry

    lax.fori_loop(0, RPT // 16, grp_d, 0)

    def grp2(i, carry):
        base = pl.multiple_of(i * 16, 16)
        ds_ = da_v[pl.ds(base, 16)]
        dd_ = db_v[pl.ds(base, 16)]
        sdr = ds_ * dd_
        rows = lax.broadcasted_iota(jnp.int32, (16,), 0) + base
        for j in range(C):
            cols = jnp.full((16,), j, jnp.int32)
            bcj = plsc.load_gather(bc_v, [cols])
            v = plsc.load_gather(tab_v, [rows, cols])
            plsc.store_scatter(tab_v, [rows, cols], v * sdr + bcj * ds_)
        return carry

    lax.fori_loop(0, RPT // 16, grp2, 0)
    pltpu.sync_copy(tab_v, ptab_sh.at[pl.ds(rbase, RPT)])
    _agg_main(ei_hbm, out_hbm, sidx_v, didx_v, rows_v, tmp_v,
              ptab_sh, acc_sh, gsem, ssem, c, s)


def _tc_mm_body(x_ref, w1_ref, w2_ref, b1_ref, p_ref, bc_ref):
    wc = jnp.dot(w1_ref[...], w2_ref[...], preferred_element_type=_f32)
    p_ref[...] = jnp.dot(x_ref[...], wc, preferred_element_type=_f32)
    bc_ref[...] = jnp.dot(b1_ref[...], w2_ref[...],
                          preferred_element_type=_f32)


_MMB = 2000


_tc_mm = pl.pallas_call(
    _tc_mm_body,
    grid=(N // _MMB,),
    in_specs=[
        pl.BlockSpec((_MMB, D), lambda i: (i, 0)),
        pl.BlockSpec((D, H), lambda i: (0, 0)),
        pl.BlockSpec((H, C), lambda i: (0, 0)),
        pl.BlockSpec((1, H), lambda i: (0, 0)),
    ],
    out_specs=(
        pl.BlockSpec((_MMB, C), lambda i: (i, 0)),
        pl.BlockSpec((1, C), lambda i: (0, 0)),
    ),
    out_shape=(
        jax.ShapeDtypeStruct((NPAD, C), _f32),
        jax.ShapeDtypeStruct((1, C), _f32),
    ),
)


RPF = NPAD // NW


@functools.partial(
    pl.kernel,
    out_type=jax.ShapeDtypeStruct((NPAD, C), _f32),
    mesh=_mesh,
    compiler_params=_sc_params,
    scratch_types=[
        pltpu.VMEM((RPF, C), _f32),
        pltpu.VMEM((RPF, C), _f32),
        pltpu.VMEM((RPF,), _f32),
        pltpu.VMEM((RPF,), _f32),
        pltpu.VMEM((16,), _f32),
    ],
)
def _fin(acc_hbm, degd_hbm, b2_hbm, out_hbm, tab_v, tab2_v, da_v, db_v, b2_v):
    c = lax.axis_index("c")
    s = lax.axis_index("s")
    wid = c * NS + s
    rbase = wid * RPF

    pltpu.sync_copy(acc_hbm.at[0, pl.ds(rbase, RPF)], tab_v)
    pltpu.sync_copy(acc_hbm.at[1, pl.ds(rbase, RPF)], tab2_v)
    pltpu.sync_copy(degd_hbm.at[pl.ds(rbase, RPF)], da_v)
    pltpu.sync_copy(degd_hbm.at[pl.ds(NPAD + rbase, RPF)], db_v)
    pltpu.sync_copy(b2_hbm, b2_v)

    def addrow(i, carry):
        tab_v[i, :] = tab_v[i, :] + tab2_v[i, :]
        return carry

    lax.fori_loop(0, RPF, addrow, 0)

    def grp_d(i, carry):
        base = pl.multiple_of(i * 16, 16)
        dsum = da_v[pl.ds(base, 16)] + db_v[pl.ds(base, 16)]
        da_v[pl.ds(base, 16)] = _rsqrt16(jnp.maximum(dsum, 1.0))
        return carry

    lax.fori_loop(0, RPF // 16, grp_d, 0)

    def grp(i, carry):
        base = pl.multiple_of(i * 16, 16)
        dd_ = da_v[pl.ds(base, 16)]
        rows = lax.broadcasted_iota(jnp.int32, (16,), 0) + base
        for j in range(C):
            cols = jnp.full((16,), j, jnp.int32)
            b2j = plsc.load_gather(b2_v, [cols])
            v = plsc.load_gather(tab_v, [rows, cols])
            plsc.store_scatter(tab_v, [rows, cols], v * dd_ + b2j)
        return carry

    lax.fori_loop(0, RPF // 16, grp, 0)
    pltpu.sync_copy(tab_v, out_hbm.at[pl.ds(rbase, RPF)])


def kernel(features, edge_index, W1, b1, W2, b2):
    degs_p, degd_p = _deg(edge_index)
    p_pad, bc = _tc_mm(features, W1, W2, b1.reshape(1, H))

    acc1 = _agg1(edge_index, p_pad, degs_p)
    acc2 = _agg2(edge_index, acc1, degs_p, degd_p, bc.reshape(C))
    return _fin(acc2, degd_p, b2)[:N]


# speedup vs baseline: 30.9095x; 1.1916x over previous
import functools

import jax
import jax.numpy as jnp
from jax import lax
from jax.experimental import pallas as pl
from jax.experimental.pallas import tpu as pltpu
from jax.experimental.pallas import tpu_sc as plsc

N = 10000
E = 320000
D = 128
H = 128
C = 16

NC = 2
NS = 16
NW = NC * NS
EPT = E // NW
RPT = 640
NPAD = NS * RPT

DW, DB = 5, 2000
AW, AB = 10, 1000

_mesh = plsc.VectorSubcoreMesh(core_axis_name="c", subcore_axis_name="s")
_f32 = jnp.float32
_sc_params = pltpu.CompilerParams(use_tc_tiling_on_sc=False, needs_layout_passes=False)


def _rsqrt16(x):
    bits = plsc.bitcast(x, jnp.int32)
    y = plsc.bitcast(jnp.int32(0x5F3759DF) - (bits >> 1), _f32)
    for _ in range(3):
        y = y * (1.5 - 0.5 * x * y * y)
    return y


@functools.partial(
    pl.kernel,
    out_type=(
        jax.ShapeDtypeStruct((NW * RPT,), _f32),
        jax.ShapeDtypeStruct((NW * RPT,), _f32),
    ),
    mesh=_mesh,
    compiler_params=_sc_params,
    scratch_types=[
        [pltpu.VMEM((DB,), jnp.int32) for _ in range(DW)],
        [pltpu.VMEM((DB,), jnp.int32) for _ in range(DW)],
        pltpu.VMEM((DB,), _f32),
        pltpu.VMEM((RPT,), _f32),
        pltpu.VMEM_SHARED((NPAD,), _f32),
        pltpu.VMEM_SHARED((NPAD,), _f32),
    ],
)
def _deg(ei_hbm, outs_hbm, outd_hbm,
         sidx_v, didx_v, ones_v, tmp_v, degs_sh, degd_sh):
    c = lax.axis_index("c")
    s = lax.axis_index("s")
    wid = c * NS + s
    ebase = wid * EPT

    def frow(i, carry):
        ones_v[pl.ds(pl.multiple_of(i * 16, 16), 16)] = jnp.ones((16,), _f32)
        return carry

    lax.fori_loop(0, DB // 16, frow, 0)

    def zrow(i, carry):
        tmp_v[pl.ds(pl.multiple_of(i * 16, 16), 16)] = jnp.zeros((16,), _f32)
        return carry

    lax.fori_loop(0, RPT // 16, zrow, 0)
    pltpu.sync_copy(tmp_v, degs_sh.at[pl.ds(s * RPT, RPT)])
    pltpu.sync_copy(tmp_v, degd_sh.at[pl.ds(s * RPT, RPT)])
    for w in range(DW):
        pltpu.sync_copy(ei_hbm.at[0, pl.ds(ebase + w * DB, DB)], sidx_v[w])
        pltpu.sync_copy(ei_hbm.at[1, pl.ds(ebase + w * DB, DB)], didx_v[w])
    plsc.subcore_barrier()
    for w in range(DW):
        pltpu.sync_copy(ones_v, degs_sh.at[sidx_v[w]], add=True)
        pltpu.sync_copy(ones_v, degd_sh.at[didx_v[w]], add=True)
    plsc.subcore_barrier()
    pltpu.sync_copy(degs_sh.at[pl.ds(s * RPT, RPT)], tmp_v)
    pltpu.sync_copy(tmp_v, outs_hbm.at[pl.ds(wid * RPT, RPT)])
    pltpu.sync_copy(degd_sh.at[pl.ds(s * RPT, RPT)], tmp_v)
    pltpu.sync_copy(tmp_v, outd_hbm.at[pl.ds(wid * RPT, RPT)])


def _agg_main(ei_hbm, out_hbm, sidx_v, didx_v, rows_v, tmp_v,
              ptab_sh, acc_sh, gsem, ssem, c, s, idx_desc):
    wid = c * NS + s
    ebase = wid * EPT

    def zrow(i, carry):
        tmp_v[i, :] = jnp.zeros((C,), _f32)
        return carry

    lax.fori_loop(0, RPT, zrow, 0)
    pltpu.sync_copy(tmp_v, acc_sh.at[pl.ds(s * RPT, RPT)])
    for d in idx_desc:
        d.wait()
    plsc.subcore_barrier()

    g_desc = [None] * AW
    s_desc = [None] * AW
    g_desc[0] = pltpu.async_copy(ptab_sh.at[sidx_v[0]], rows_v[0], gsem[0])
    for w in range(AW):
        g_desc[w].wait()
        s_desc[w] = pltpu.async_copy(
            rows_v[w % 2], acc_sh.at[didx_v[w]], ssem[w % 2], add=True)
        if w + 1 < AW:
            if w >= 1:
                s_desc[w - 1].wait()
            g_desc[w + 1] = pltpu.async_copy(
                ptab_sh.at[sidx_v[w + 1]], rows_v[(w + 1) % 2],
                gsem[(w + 1) % 2])
    s_desc[AW - 1].wait()
    plsc.subcore_barrier()
    pltpu.sync_copy(acc_sh.at[pl.ds(s * RPT, RPT)], tmp_v)
    pltpu.sync_copy(tmp_v, out_hbm.at[c, pl.ds(s * RPT, RPT)])


_AGG_SCRATCH = [
    [pltpu.VMEM((AB,), jnp.int32) for _ in range(AW)],
    [pltpu.VMEM((AB,), jnp.int32) for _ in range(AW)],
    [pltpu.VMEM((AB, C), _f32) for _ in range(2)],
    pltpu.VMEM((RPT, C), _f32),
    pltpu.VMEM((RPT,), _f32),
    pltpu.VMEM((RPT,), _f32),
    pltpu.VMEM_SHARED((NPAD, C), _f32),
    pltpu.VMEM_SHARED((NPAD, C), _f32),
    [pltpu.SemaphoreType.DMA for _ in range(2)],
    [pltpu.SemaphoreType.DMA for _ in range(2)],
]


@functools.partial(
    pl.kernel,
    out_type=jax.ShapeDtypeStruct((NC, NPAD, C), _f32),
    mesh=_mesh,
    compiler_params=_sc_params,
    scratch_types=[pltpu.VMEM((RPT, C), _f32)] + _AGG_SCRATCH,
)
def _agg1(ei_hbm, p_hbm, degs_hbm, out_hbm,
          tab_v, sidx_v, didx_v, rows_v, tmp_v, da_v, db_v,
          ptab_sh, acc_sh, gsem, ssem):
    c = lax.axis_index("c")
    s = lax.axis_index("s")
    rbase = s * RPT
    ebase = (c * NS + s) * EPT
    idx_desc = [
        pltpu.async_copy(ei_hbm.at[0, pl.ds(ebase + w * AB, AB)], sidx_v[w],
                         gsem[0]) for w in range(AW)
    ] + [
        pltpu.async_copy(ei_hbm.at[1, pl.ds(ebase + w * AB, AB)], didx_v[w],
                         gsem[1]) for w in range(AW)
    ]

    pltpu.sync_copy(p_hbm.at[pl.ds(rbase, RPT)], tab_v)
    pltpu.sync_copy(degs_hbm.at[pl.ds(rbase, RPT)], da_v)
    pltpu.sync_copy(degs_hbm.at[pl.ds(NPAD + rbase, RPT)], db_v)

    def grp(i, carry):
        base = pl.multiple_of(i * 16, 16)
        dsum = da_v[pl.ds(base, 16)] + db_v[pl.ds(base, 16)]
        y = _rsqrt16(jnp.maximum(dsum, 1.0))
        rows = lax.broadcasted_iota(jnp.int32, (16,), 0) + base
        for j in range(C):
            cols = jnp.full((16,), j, jnp.int32)
            v = plsc.load_gather(tab_v, [rows, cols])
            plsc.store_scatter(tab_v, [rows, cols], v * y)
        return carry

    lax.fori_loop(0, RPT // 16, grp, 0)
    pltpu.sync_copy(tab_v, ptab_sh.at[pl.ds(rbase, RPT)])
    _agg_main(ei_hbm, out_hbm, sidx_v, didx_v, rows_v, tmp_v,
              ptab_sh, acc_sh, gsem, ssem, c, s, idx_desc)


@functools.partial(
    pl.kernel,
    out_type=jax.ShapeDtypeStruct((NC, NPAD, C), _f32),
    mesh=_mesh,
    compiler_params=_sc_params,
    scratch_types=[pltpu.VMEM((RPT, C), _f32), pltpu.VMEM((RPT, C), _f32),
                   pltpu.VMEM((16,), _f32),
                   pltpu.VMEM((RPT,), _f32)] + _AGG_SCRATCH,
)
def _agg2(ei_hbm, q_hbm, degs_hbm, degd_hbm, bc_hbm, out_hbm,
          tab_v, tab2_v, bc_v, dc_v, sidx_v, didx_v, rows_v, tmp_v,
          da_v, db_v, ptab_sh, acc_sh, gsem, ssem):
    c = lax.axis_index("c")
    s = lax.axis_index("s")
    rbase = s * RPT
    ebase = (c * NS + s) * EPT
    idx_desc = [
        pltpu.async_copy(ei_hbm.at[0, pl.ds(ebase + w * AB, AB)], sidx_v[w],
                         gsem[0]) for w in range(AW)
    ] + [
        pltpu.async_copy(ei_hbm.at[1, pl.ds(ebase + w * AB, AB)], didx_v[w],
                         gsem[1]) for w in range(AW)
    ]

    pltpu.sync_copy(q_hbm.at[0, pl.ds(rbase, RPT)], tab_v)
    pltpu.sync_copy(q_hbm.at[1, pl.ds(rbase, RPT)], tab2_v)
    pltpu.sync_copy(bc_hbm, bc_v)

    pltpu.sync_copy(degs_hbm.at[pl.ds(rbase, RPT)], da_v)
    pltpu.sync_copy(degs_hbm.at[pl.ds(NPAD + rbase, RPT)], db_v)

    def grp_s(i, carry):
        base = pl.multiple_of(i * 16, 16)
        dsum = da_v[pl.ds(base, 16)] + db_v[pl.ds(base, 16)]
        da_v[pl.ds(base, 16)] = _rsqrt16(jnp.maximum(dsum, 1.0))
        return carry

    lax.fori_loop(0, RPT // 16, grp_s, 0)
    pltpu.sync_copy(degd_hbm.at[pl.ds(rbase, RPT)], db_v)
    pltpu.sync_copy(degd_hbm.at[pl.ds(NPAD + rbase, RPT)], dc_v)

    def grp_d(i, carry):
        base = pl.multiple_of(i * 16, 16)
        dsum = db_v[pl.ds(base, 16)] + dc_v[pl.ds(base, 16)]
        db_v[pl.ds(base, 16)] = _rsqrt16(jnp.maximum(dsum, 1.0))
        return carry

    lax.fori_loop(0, RPT // 16, grp_d, 0)

    def grp2(i, carry):
        base = pl.multiple_of(i * 16, 16)
        ds_ = da_v[pl.ds(base, 16)]
        dd_ = db_v[pl.ds(base, 16)]
        sdr = ds_ * dd_
        rows = lax.broadcasted_iota(jnp.int32, (16,), 0) + base
        for j in range(C):
            cols = jnp.full((16,), j, jnp.int32)
            bcj = plsc.load_gather(bc_v, [cols])
            v = (plsc.load_gather(tab_v, [rows, cols])
                 + plsc.load_gather(tab2_v, [rows, cols]))
            plsc.store_scatter(tab_v, [rows, cols], v * sdr + bcj * ds_)
        return carry

    lax.fori_loop(0, RPT // 16, grp2, 0)
    pltpu.sync_copy(tab_v, ptab_sh.at[pl.ds(rbase, RPT)])
    _agg_main(ei_hbm, out_hbm, sidx_v, didx_v, rows_v, tmp_v,
              ptab_sh, acc_sh, gsem, ssem, c, s, idx_desc)


def _tc_mm_body(x_ref, w1_ref, w2_ref, b1_ref, p_ref, bc_ref):
    wc = jnp.dot(w1_ref[...], w2_ref[...], preferred_element_type=_f32)
    p_ref[...] = jnp.dot(x_ref[...], wc, preferred_element_type=_f32)
    bc_ref[...] = jnp.dot(b1_ref[...], w2_ref[...],
                          preferred_element_type=_f32)


_MMB = 2000


_tc_mm = pl.pallas_call(
    _tc_mm_body,
    grid=(N // _MMB,),
    in_specs=[
        pl.BlockSpec((_MMB, D), lambda i: (i, 0)),
        pl.BlockSpec((D, H), lambda i: (0, 0)),
        pl.BlockSpec((H, C), lambda i: (0, 0)),
        pl.BlockSpec((1, H), lambda i: (0, 0)),
    ],
    out_specs=(
        pl.BlockSpec((_MMB, C), lambda i: (i, 0)),
        pl.BlockSpec((1, C), lambda i: (0, 0)),
    ),
    out_shape=(
        jax.ShapeDtypeStruct((NPAD, C), _f32),
        jax.ShapeDtypeStruct((1, C), _f32),
    ),
)


RPF = NPAD // NW


@functools.partial(
    pl.kernel,
    out_type=jax.ShapeDtypeStruct((N, C), _f32),
    mesh=_mesh,
    compiler_params=_sc_params,
    scratch_types=[
        pltpu.VMEM((RPF, C), _f32),
        pltpu.VMEM((RPF, C), _f32),
        pltpu.VMEM((RPF,), _f32),
        pltpu.VMEM((RPF,), _f32),
        pltpu.VMEM((16,), _f32),
    ],
)
def _fin(acc_hbm, degd_hbm, b2_hbm, out_hbm, tab_v, tab2_v, da_v, db_v, b2_v):
    c = lax.axis_index("c")
    s = lax.axis_index("s")
    wid = c * NS + s
    rbase = wid * RPF

    pltpu.sync_copy(acc_hbm.at[0, pl.ds(rbase, RPF)], tab_v)
    pltpu.sync_copy(acc_hbm.at[1, pl.ds(rbase, RPF)], tab2_v)
    pltpu.sync_copy(degd_hbm.at[pl.ds(rbase, RPF)], da_v)
    pltpu.sync_copy(degd_hbm.at[pl.ds(NPAD + rbase, RPF)], db_v)
    pltpu.sync_copy(b2_hbm, b2_v)

    def addrow(i, carry):
        tab_v[i, :] = tab_v[i, :] + tab2_v[i, :]
        return carry

    lax.fori_loop(0, RPF, addrow, 0)

    def grp_d(i, carry):
        base = pl.multiple_of(i * 16, 16)
        dsum = da_v[pl.ds(base, 16)] + db_v[pl.ds(base, 16)]
        da_v[pl.ds(base, 16)] = _rsqrt16(jnp.maximum(dsum, 1.0))
        return carry

    lax.fori_loop(0, RPF // 16, grp_d, 0)

    def grp(i, carry):
        base = pl.multiple_of(i * 16, 16)
        dd_ = da_v[pl.ds(base, 16)]
        rows = lax.broadcasted_iota(jnp.int32, (16,), 0) + base
        for j in range(C):
            cols = jnp.full((16,), j, jnp.int32)
            b2j = plsc.load_gather(b2_v, [cols])
            v = plsc.load_gather(tab_v, [rows, cols])
            plsc.store_scatter(tab_v, [rows, cols], v * dd_ + b2j)
        return carry

    lax.fori_loop(0, RPF // 16, grp, 0)
    last = N - (NW - 1) * RPF

    @pl.when(wid < NW - 1)
    def _():
        pltpu.sync_copy(tab_v, out_hbm.at[pl.ds(rbase, RPF)])

    @pl.when(wid == NW - 1)
    def _():
        pltpu.sync_copy(tab_v.at[pl.ds(0, last)],
                        out_hbm.at[pl.ds((NW - 1) * RPF, last)])


def kernel(features, edge_index, W1, b1, W2, b2):
    degs_p, degd_p = _deg(edge_index)
    p_pad, bc = _tc_mm(features, W1, W2, b1.reshape(1, H))

    acc1 = _agg1(edge_index, p_pad, degs_p)
    acc2 = _agg2(edge_index, acc1, degs_p, degd_p, bc.reshape(C))
    return _fin(acc2, degd_p, b2)


# speedup vs baseline: 32.4549x; 1.0500x over previous
import functools

import jax
import jax.numpy as jnp
from jax import lax
from jax.experimental import pallas as pl
from jax.experimental.pallas import tpu as pltpu
from jax.experimental.pallas import tpu_sc as plsc

N = 10000
E = 320000
D = 128
H = 128
C = 16

NC = 2
NS = 16
NW = NC * NS
EPT = E // NW
RPT = 640
NPAD = NS * RPT

DW, DB = 5, 2000
AW, AB = 10, 1000

_mesh = plsc.VectorSubcoreMesh(core_axis_name="c", subcore_axis_name="s")
_f32 = jnp.float32
_sc_params = pltpu.CompilerParams(use_tc_tiling_on_sc=False, needs_layout_passes=False)


def _rsqrt16(x):
    bits = plsc.bitcast(x, jnp.int32)
    y = plsc.bitcast(jnp.int32(0x5F3759DF) - (bits >> 1), _f32)
    for _ in range(3):
        y = y * (1.5 - 0.5 * x * y * y)
    return y


@functools.partial(
    pl.kernel,
    out_type=(
        jax.ShapeDtypeStruct((NW * RPT,), _f32),
        jax.ShapeDtypeStruct((NW * RPT,), _f32),
    ),
    mesh=_mesh,
    compiler_params=_sc_params,
    scratch_types=[
        [pltpu.VMEM((DB,), jnp.int32) for _ in range(DW)],
        [pltpu.VMEM((DB,), jnp.int32) for _ in range(DW)],
        pltpu.VMEM((DB,), _f32),
        pltpu.VMEM((RPT,), _f32),
        pltpu.VMEM((RPT,), _f32),
        pltpu.VMEM_SHARED((NPAD,), _f32),
        pltpu.VMEM_SHARED((NPAD,), _f32),
        pltpu.SemaphoreType.DMA,
        pltpu.SemaphoreType.DMA,
    ],
)
def _deg(ei_hbm, outs_hbm, outd_hbm,
         sidx_v, didx_v, ones_v, tmp_v, tmp2_v, degs_sh, degd_sh,
         isem, asem):
    c = lax.axis_index("c")
    s = lax.axis_index("s")
    wid = c * NS + s
    ebase = wid * EPT

    idx_desc = [
        pltpu.async_copy(ei_hbm.at[0, pl.ds(ebase + w * DB, DB)], sidx_v[w],
                         isem) for w in range(DW)
    ] + [
        pltpu.async_copy(ei_hbm.at[1, pl.ds(ebase + w * DB, DB)], didx_v[w],
                         isem) for w in range(DW)
    ]

    def frow(i, carry):
        ones_v[pl.ds(pl.multiple_of(i * 16, 16), 16)] = jnp.ones((16,), _f32)
        return carry

    lax.fori_loop(0, DB // 16, frow, 0)

    def zrow(i, carry):
        tmp_v[pl.ds(pl.multiple_of(i * 16, 16), 16)] = jnp.zeros((16,), _f32)
        return carry

    lax.fori_loop(0, RPT // 16, zrow, 0)
    pltpu.sync_copy(tmp_v, degs_sh.at[pl.ds(s * RPT, RPT)])
    pltpu.sync_copy(tmp_v, degd_sh.at[pl.ds(s * RPT, RPT)])
    for d in idx_desc:
        d.wait()
    plsc.subcore_barrier()
    for w in range(DW):
        pltpu.sync_copy(ones_v, degs_sh.at[sidx_v[w]], add=True)
        pltpu.sync_copy(ones_v, degd_sh.at[didx_v[w]], add=True)
    plsc.subcore_barrier()
    pltpu.sync_copy(degs_sh.at[pl.ds(s * RPT, RPT)], tmp_v)
    pltpu.sync_copy(degd_sh.at[pl.ds(s * RPT, RPT)], tmp2_v)
    pltpu.sync_copy(tmp_v, outs_hbm.at[pl.ds(wid * RPT, RPT)])
    pltpu.sync_copy(tmp2_v, outd_hbm.at[pl.ds(wid * RPT, RPT)])


def _agg_main(ei_hbm, out_hbm, sidx_v, didx_v, rows_v, tmp_v,
              ptab_sh, acc_sh, gsem, ssem, c, s, idx_desc):
    wid = c * NS + s
    ebase = wid * EPT

    def zrow(i, carry):
        tmp_v[i, :] = jnp.zeros((C,), _f32)
        return carry

    lax.fori_loop(0, RPT, zrow, 0)
    pltpu.sync_copy(tmp_v, acc_sh.at[pl.ds(s * RPT, RPT)])
    for d in idx_desc:
        d.wait()
    plsc.subcore_barrier()

    g_desc = [None] * AW
    s_desc = [None] * AW
    g_desc[0] = pltpu.async_copy(ptab_sh.at[sidx_v[0]], rows_v[0], gsem[0])
    for w in range(AW):
        g_desc[w].wait()
        s_desc[w] = pltpu.async_copy(
            rows_v[w % 2], acc_sh.at[didx_v[w]], ssem[w % 2], add=True)
        if w + 1 < AW:
            if w >= 1:
                s_desc[w - 1].wait()
            g_desc[w + 1] = pltpu.async_copy(
                ptab_sh.at[sidx_v[w + 1]], rows_v[(w + 1) % 2],
                gsem[(w + 1) % 2])
    s_desc[AW - 1].wait()
    plsc.subcore_barrier()
    pltpu.sync_copy(acc_sh.at[pl.ds(s * RPT, RPT)], tmp_v)
    pltpu.sync_copy(tmp_v, out_hbm.at[c, pl.ds(s * RPT, RPT)])


_AGG_SCRATCH = [
    [pltpu.VMEM((AB,), jnp.int32) for _ in range(AW)],
    [pltpu.VMEM((AB,), jnp.int32) for _ in range(AW)],
    [pltpu.VMEM((AB, C), _f32) for _ in range(2)],
    pltpu.VMEM((RPT, C), _f32),
    pltpu.VMEM((RPT,), _f32),
    pltpu.VMEM((RPT,), _f32),
    pltpu.VMEM_SHARED((NPAD, C), _f32),
    pltpu.VMEM_SHARED((NPAD, C), _f32),
    [pltpu.SemaphoreType.DMA for _ in range(2)],
    [pltpu.SemaphoreType.DMA for _ in range(2)],
]


@functools.partial(
    pl.kernel,
    out_type=jax.ShapeDtypeStruct((NC, NPAD, C), _f32),
    mesh=_mesh,
    compiler_params=_sc_params,
    scratch_types=[pltpu.VMEM((RPT, C), _f32)] + _AGG_SCRATCH,
)
def _agg1(ei_hbm, p_hbm, degs_hbm, out_hbm,
          tab_v, sidx_v, didx_v, rows_v, tmp_v, da_v, db_v,
          ptab_sh, acc_sh, gsem, ssem):
    c = lax.axis_index("c")
    s = lax.axis_index("s")
    rbase = s * RPT
    ebase = (c * NS + s) * EPT
    idx_desc = [
        pltpu.async_copy(ei_hbm.at[0, pl.ds(ebase + w * AB, AB)], sidx_v[w],
                         gsem[0]) for w in range(AW)
    ] + [
        pltpu.async_copy(ei_hbm.at[1, pl.ds(ebase + w * AB, AB)], didx_v[w],
                         gsem[1]) for w in range(AW)
    ]

    pltpu.sync_copy(p_hbm.at[pl.ds(rbase, RPT)], tab_v)
    pltpu.sync_copy(degs_hbm.at[pl.ds(rbase, RPT)], da_v)
    pltpu.sync_copy(degs_hbm.at[pl.ds(NPAD + rbase, RPT)], db_v)

    def grp(i, carry):
        base = pl.multiple_of(i * 16, 16)
        dsum = da_v[pl.ds(base, 16)] + db_v[pl.ds(base, 16)]
        y = _rsqrt16(jnp.maximum(dsum, 1.0))
        rows = lax.broadcasted_iota(jnp.int32, (16,), 0) + base
        for j in range(C):
            cols = jnp.full((16,), j, jnp.int32)
            v = plsc.load_gather(tab_v, [rows, cols])
            plsc.store_scatter(tab_v, [rows, cols], v * y)
        return carry

    lax.fori_loop(0, RPT // 16, grp, 0)
    pltpu.sync_copy(tab_v, ptab_sh.at[pl.ds(rbase, RPT)])
    _agg_main(ei_hbm, out_hbm, sidx_v, didx_v, rows_v, tmp_v,
              ptab_sh, acc_sh, gsem, ssem, c, s, idx_desc)


@functools.partial(
    pl.kernel,
    out_type=jax.ShapeDtypeStruct((NC, NPAD, C), _f32),
    mesh=_mesh,
    compiler_params=_sc_params,
    scratch_types=[pltpu.VMEM((RPT, C), _f32), pltpu.VMEM((RPT, C), _f32),
                   pltpu.VMEM((16,), _f32),
                   pltpu.VMEM((RPT,), _f32),
                   pltpu.VMEM((RPT,), _f32)] + _AGG_SCRATCH,
)
def _agg2(ei_hbm, q_hbm, degs_hbm, degd_hbm, bc_hbm, out_hbm,
          tab_v, tab2_v, bc_v, dc_v, dd_v, sidx_v, didx_v, rows_v, tmp_v,
          da_v, db_v, ptab_sh, acc_sh, gsem, ssem):
    c = lax.axis_index("c")
    s = lax.axis_index("s")
    rbase = s * RPT
    ebase = (c * NS + s) * EPT
    idx_desc = [
        pltpu.async_copy(ei_hbm.at[0, pl.ds(ebase + w * AB, AB)], sidx_v[w],
                         gsem[0]) for w in range(AW)
    ] + [
        pltpu.async_copy(ei_hbm.at[1, pl.ds(ebase + w * AB, AB)], didx_v[w],
                         gsem[1]) for w in range(AW)
    ]

    pltpu.sync_copy(q_hbm.at[0, pl.ds(rbase, RPT)], tab_v)
    pltpu.sync_copy(q_hbm.at[1, pl.ds(rbase, RPT)], tab2_v)
    pltpu.sync_copy(bc_hbm, bc_v)
    pltpu.sync_copy(degs_hbm.at[pl.ds(rbase, RPT)], da_v)
    pltpu.sync_copy(degs_hbm.at[pl.ds(NPAD + rbase, RPT)], db_v)
    pltpu.sync_copy(degd_hbm.at[pl.ds(rbase, RPT)], dc_v)
    pltpu.sync_copy(degd_hbm.at[pl.ds(NPAD + rbase, RPT)], dd_v)

    def grp_s(i, carry):
        base = pl.multiple_of(i * 16, 16)
        dsum_s = da_v[pl.ds(base, 16)] + db_v[pl.ds(base, 16)]
        dsum_d = dc_v[pl.ds(base, 16)] + dd_v[pl.ds(base, 16)]
        da_v[pl.ds(base, 16)] = _rsqrt16(jnp.maximum(dsum_s, 1.0))
        db_v[pl.ds(base, 16)] = _rsqrt16(jnp.maximum(dsum_d, 1.0))
        return carry

    lax.fori_loop(0, RPT // 16, grp_s, 0)

    def grp2(i, carry):
        base = pl.multiple_of(i * 16, 16)
        ds_ = da_v[pl.ds(base, 16)]
        dd_ = db_v[pl.ds(base, 16)]
        sdr = ds_ * dd_
        rows = lax.broadcasted_iota(jnp.int32, (16,), 0) + base
        for j in range(C):
            cols = jnp.full((16,), j, jnp.int32)
            bcj = plsc.load_gather(bc_v, [cols])
            v = (plsc.load_gather(tab_v, [rows, cols])
                 + plsc.load_gather(tab2_v, [rows, cols]))
            plsc.store_scatter(tab_v, [rows, cols], v * sdr + bcj * ds_)
        return carry

    lax.fori_loop(0, RPT // 16, grp2, 0)
    pltpu.sync_copy(tab_v, ptab_sh.at[pl.ds(rbase, RPT)])
    _agg_main(ei_hbm, out_hbm, sidx_v, didx_v, rows_v, tmp_v,
              ptab_sh, acc_sh, gsem, ssem, c, s, idx_desc)


def _tc_mm_body(x_ref, w1_ref, w2_ref, b1_ref, p_ref, bc_ref):
    wc = jnp.dot(w1_ref[...], w2_ref[...], preferred_element_type=_f32)
    p_ref[...] = jnp.dot(x_ref[...], wc, preferred_element_type=_f32)
    bc_ref[...] = jnp.dot(b1_ref[...], w2_ref[...],
                          preferred_element_type=_f32)


_MMB = 2000


_tc_mm = pl.pallas_call(
    _tc_mm_body,
    grid=(N // _MMB,),
    in_specs=[
        pl.BlockSpec((_MMB, D), lambda i: (i, 0)),
        pl.BlockSpec((D, H), lambda i: (0, 0)),
        pl.BlockSpec((H, C), lambda i: (0, 0)),
        pl.BlockSpec((1, H), lambda i: (0, 0)),
    ],
    out_specs=(
        pl.BlockSpec((_MMB, C), lambda i: (i, 0)),
        pl.BlockSpec((1, C), lambda i: (0, 0)),
    ),
    out_shape=(
        jax.ShapeDtypeStruct((NPAD, C), _f32),
        jax.ShapeDtypeStruct((1, C), _f32),
    ),
)


RPF = NPAD // NW


@functools.partial(
    pl.kernel,
    out_type=jax.ShapeDtypeStruct((N, C), _f32),
    mesh=_mesh,
    compiler_params=_sc_params,
    scratch_types=[
        pltpu.VMEM((RPF, C), _f32),
        pltpu.VMEM((RPF, C), _f32),
        pltpu.VMEM((RPF,), _f32),
        pltpu.VMEM((RPF,), _f32),
        pltpu.VMEM((16,), _f32),
        pltpu.SemaphoreType.DMA,
    ],
)
def _fin(acc_hbm, degd_hbm, b2_hbm, out_hbm, tab_v, tab2_v, da_v, db_v,
         b2_v, fsem):
    c = lax.axis_index("c")
    s = lax.axis_index("s")
    wid = c * NS + s
    rbase = wid * RPF

    pltpu.sync_copy(acc_hbm.at[0, pl.ds(rbase, RPF)], tab_v)
    pltpu.sync_copy(acc_hbm.at[1, pl.ds(rbase, RPF)], tab2_v)
    pltpu.sync_copy(degd_hbm.at[pl.ds(rbase, RPF)], da_v)
    pltpu.sync_copy(degd_hbm.at[pl.ds(NPAD + rbase, RPF)], db_v)
    pltpu.sync_copy(b2_hbm, b2_v)

    def addrow(i, carry):
        tab_v[i, :] = tab_v[i, :] + tab2_v[i, :]
        return carry

    lax.fori_loop(0, RPF, addrow, 0)

    def grp_d(i, carry):
        base = pl.multiple_of(i * 16, 16)
        dsum = da_v[pl.ds(base, 16)] + db_v[pl.ds(base, 16)]
        da_v[pl.ds(base, 16)] = _rsqrt16(jnp.maximum(dsum, 1.0))
        return carry

    lax.fori_loop(0, RPF // 16, grp_d, 0)

    def grp(i, carry):
        base = pl.multiple_of(i * 16, 16)
        dd_ = da_v[pl.ds(base, 16)]
        rows = lax.broadcasted_iota(jnp.int32, (16,), 0) + base
        for j in range(C):
            cols = jnp.full((16,), j, jnp.int32)
            b2j = plsc.load_gather(b2_v, [cols])
            v = plsc.load_gather(tab_v, [rows, cols])
            plsc.store_scatter(tab_v, [rows, cols], v * dd_ + b2j)
        return carry

    lax.fori_loop(0, RPF // 16, grp, 0)
    last = N - (NW - 1) * RPF

    @pl.when(wid < NW - 1)
    def _():
        pltpu.sync_copy(tab_v, out_hbm.at[pl.ds(rbase, RPF)])

    @pl.when(wid == NW - 1)
    def _():
        pltpu.sync_copy(tab_v.at[pl.ds(0, last)],
                        out_hbm.at[pl.ds((NW - 1) * RPF, last)])


def kernel(features, edge_index, W1, b1, W2, b2):
    degs_p, degd_p = _deg(edge_index)
    p_pad, bc = _tc_mm(features, W1, W2, b1.reshape(1, H))

    acc1 = _agg1(edge_index, p_pad, degs_p)
    acc2 = _agg2(edge_index, acc1, degs_p, degd_p, bc.reshape(C))
    return _fin(acc2, degd_p, b2)
